# baseline (speedup 1.0000x reference)
import math

import jax
import jax.numpy as jnp
from jax.experimental import pallas as pl

BS = 64
N = 500
NP = 512
INPUT_DIM = 6
HIDDEN = 32
HEADS = 2
OUT_DIM = 10
RADIUS = 0.1
D_MODEL = HIDDEN * HEADS
NEG = -jnp.inf
HE = HIDDEN + 1
GPB = 2


def _attn(x_bf, mask_add, wq, bq, wk, bk, wv, bv, normalize=True):
    outs = []
    for h in range(HEADS):
        qh = (jax.lax.dot_general(
            x_bf, wq[h][:], (((1,), (0,)), ((), ())),
            preferred_element_type=jnp.float32) + bq[h][:]).astype(jnp.bfloat16)
        kh = (jax.lax.dot_general(
            x_bf, wk[h][:], (((1,), (0,)), ((), ())),
            preferred_element_type=jnp.float32) + bk[h][:]).astype(jnp.bfloat16)
        ve = (jax.lax.dot_general(
            x_bf, wv[h][:], (((1,), (0,)), ((), ())),
            preferred_element_type=jnp.float32) + bv[h][:]).astype(jnp.bfloat16)
        al = jax.lax.dot_general(
            qh, kh, (((1,), (1,)), ((), ())),
            preferred_element_type=jnp.float32,
        ).astype(jnp.bfloat16)
        al = al + mask_add
        amax = jnp.max(al, axis=1, keepdims=True)
        amax = jnp.maximum(amax, jnp.bfloat16(-1e20))
        ex = jnp.exp(al - amax)
        oh = jax.lax.dot_general(
            ex, ve, (((1,), (0,)), ((), ())),
            preferred_element_type=jnp.float32,
        )
        if normalize:
            den = oh[:, HIDDEN:HE] + 1e-16
            outs.append(oh[:, :HIDDEN] / den)
        else:
            outs.append(oh)
    if normalize:
        return jnp.concatenate(outs, axis=1)
    return outs


def _body(feats_ref, pos_ref, posT_ref, onehot_ref,
          W1_ref, b1_ref, W2_ref, b2_ref,
          q10_ref, bq10_ref, q11_ref, bq11_ref,
          k10_ref, bk10_ref, k11_ref, bk11_ref,
          v10_ref, bv10_ref, v11_ref, bv11_ref,
          q20_ref, bq20_ref, q21_ref, bq21_ref,
          k20_ref, bk20_ref, k21_ref, bk21_ref,
          v20_ref, bv20_ref, v21_ref, bv21_ref,
          oW_ref, ob_ref, eyeB_ref, out_ref):
    for g in range(GPB):
        feats = feats_ref[g]
        h = jnp.maximum(feats @ W1_ref[:] + b1_ref[:], 0.0)
        h = jnp.maximum(h @ W2_ref[:] + b2_ref[:], 0.0)

        pos = pos_ref[g]
        posT = posT_ref[g]
        dx = pos[:, 0:1] - posT[0:1, :]
        dy = pos[:, 1:2] - posT[1:2, :]
        d2 = dx * dx + dy * dy
        t = jnp.maximum(d2 + eyeB_ref[:], 0.0)
        mask_add = (t * -1e38).astype(jnp.bfloat16)

        h = jnp.maximum(
            _attn(h.astype(jnp.bfloat16), mask_add,
                  (q10_ref, q11_ref), (bq10_ref, bq11_ref),
                  (k10_ref, k11_ref), (bk10_ref, bk11_ref),
                  (v10_ref, v11_ref), (bv10_ref, bv11_ref)), 0.0)
        ohs = _attn(h.astype(jnp.bfloat16), mask_add,
                    (q20_ref, q21_ref), (bq20_ref, bq21_ref),
                    (k20_ref, k21_ref), (bk20_ref, bk21_ref),
                    (v20_ref, v21_ref), (bv20_ref, bv21_ref),
                    normalize=False)
        embs = []
        for oh in ohs:
            gh = jax.lax.dot_general(
                onehot_ref[g], jnp.maximum(oh, 0.0), (((1,), (0,)), ((), ())),
                preferred_element_type=jnp.float32,
            )
            embs.append(gh[:, :HIDDEN] / (gh[:, HIDDEN:HE] + 1e-16))
        emb = jnp.concatenate(embs, axis=1)
        out_ref[g] = emb @ oW_ref[:] + ob_ref[:]


def _split_heads(Wq, bq, Wk, bk, Wv, bv):
    scale = 1.0 / math.sqrt(HIDDEN)
    out = []
    for h in range(HEADS):
        sl = slice(h * HIDDEN, (h + 1) * HIDDEN)
        wqh = (Wq[:, sl] * scale).astype(jnp.bfloat16)
        bqh = (bq[sl] * scale)[None, :]
        wkh = Wk[:, sl].astype(jnp.bfloat16)
        bkh = bk[sl][None, :]
        wvh = jnp.concatenate(
            [Wv[:, sl], jnp.zeros((Wv.shape[0], 1), jnp.float32)], axis=1
        ).astype(jnp.bfloat16)
        bvh = jnp.concatenate([bv[sl], jnp.ones((1,), jnp.float32)])[None, :]
        out.extend([wqh, bqh, wkh, bkh, wvh, bvh])
    return [out[i] for i in (0, 1, 6, 7, 2, 3, 8, 9, 4, 5, 10, 11)]


@jax.jit
def kernel(obs, enc_W1, enc_b1, enc_W2, enc_b2,
           c1_Wq, c1_bq, c1_Wk, c1_bk, c1_Wv, c1_bv,
           c2_Wq, c2_bq, c2_Wk, c2_bk, c2_Wv, c2_bv,
           out_W, out_b):
    node = obs[:, :N * (2 + INPUT_DIM)].reshape(BS, N, 2 + INPUT_DIM)
    pos = node[:, :, :2]
    feats = node[:, :, 2:]
    pos_p = jnp.pad(pos, ((0, 0), (0, NP - N), (0, 0)), constant_values=1e6)
    feats_p = jnp.pad(feats, ((0, 0), (0, NP - N), (0, 2)))
    posT_p = jnp.swapaxes(pos_p, 1, 2)

    agent = jnp.clip(obs[:, -1], 0, N - 1).astype(jnp.int32)
    onehot = jax.nn.one_hot(agent, NP, dtype=jnp.float32)[:, None, :]

    W1 = jnp.pad(enc_W1, ((0, 2), (0, 0)))
    b1 = enc_b1[None, :]
    b2 = enc_b2[None, :]
    ob = out_b[None, :]
    l1 = _split_heads(c1_Wq, c1_bq, c1_Wk, c1_bk, c1_Wv, c1_bv)
    l2 = _split_heads(c2_Wq, c2_bq, c2_Wk, c2_bk, c2_Wv, c2_bv)

    def fixed(a):
        nd = a.ndim
        return pl.BlockSpec(a.shape, lambda b: (0,) * nd)

    r2 = jnp.float32(RADIUS * RADIUS)
    eyeB = jnp.full((NP, NP), -r2, jnp.float32) + jnp.eye(NP, dtype=jnp.float32) * (1.0 + r2)
    weights = [W1, b1, enc_W2, b2] + l1 + l2 + [out_W, ob, eyeB]
    in_specs = [
        pl.BlockSpec((GPB, NP, 8), lambda b: (b, 0, 0)),
        pl.BlockSpec((GPB, NP, 2), lambda b: (b, 0, 0)),
        pl.BlockSpec((GPB, 2, NP), lambda b: (b, 0, 0)),
        pl.BlockSpec((GPB, 1, NP), lambda b: (b, 0, 0)),
    ] + [fixed(w) for w in weights]

    out = pl.pallas_call(
        _body,
        grid=(BS // GPB,),
        in_specs=in_specs,
        out_specs=pl.BlockSpec((GPB, 1, OUT_DIM), lambda b: (b, 0, 0)),
        out_shape=jax.ShapeDtypeStruct((BS, 1, OUT_DIM), jnp.float32),
    )(feats_p, pos_p, posT_p, onehot, *weights)
    return out[:, 0, :]


# speedup vs baseline: 1.0104x; 1.0104x over previous
import math

import jax
import jax.numpy as jnp
from jax.experimental import pallas as pl

BS = 64
N = 500
NP = 512
INPUT_DIM = 6
HIDDEN = 32
HEADS = 2
OUT_DIM = 10
RADIUS = 0.1
D_MODEL = HIDDEN * HEADS
NEG = -jnp.inf
HE = HIDDEN + 1
GPB = 4


def _attn(x_bf, mask_add, wq, bq, wk, bk, wv, bv, normalize=True):
    outs = []
    for h in range(HEADS):
        qh = (jax.lax.dot_general(
            x_bf, wq[h][:], (((1,), (0,)), ((), ())),
            preferred_element_type=jnp.float32) + bq[h][:]).astype(jnp.bfloat16)
        kh = (jax.lax.dot_general(
            x_bf, wk[h][:], (((1,), (0,)), ((), ())),
            preferred_element_type=jnp.float32) + bk[h][:]).astype(jnp.bfloat16)
        ve = (jax.lax.dot_general(
            x_bf, wv[h][:], (((1,), (0,)), ((), ())),
            preferred_element_type=jnp.float32) + bv[h][:]).astype(jnp.bfloat16)
        al = jax.lax.dot_general(
            qh, kh, (((1,), (1,)), ((), ())),
            preferred_element_type=jnp.float32,
        ).astype(jnp.bfloat16)
        al = al + mask_add
        amax = jnp.max(al, axis=1, keepdims=True)
        amax = jnp.maximum(amax, jnp.bfloat16(-1e20))
        ex = jnp.exp(al - amax)
        oh = jax.lax.dot_general(
            ex, ve, (((1,), (0,)), ((), ())),
            preferred_element_type=jnp.float32,
        )
        if normalize:
            den = oh[:, HIDDEN:HE] + 1e-16
            outs.append(oh[:, :HIDDEN] / den)
        else:
            outs.append(oh)
    if normalize:
        return jnp.concatenate(outs, axis=1)
    return outs


def _body(feats_ref, pos_ref, posT_ref, onehot_ref,
          W1_ref, b1_ref, W2_ref, b2_ref,
          q10_ref, bq10_ref, q11_ref, bq11_ref,
          k10_ref, bk10_ref, k11_ref, bk11_ref,
          v10_ref, bv10_ref, v11_ref, bv11_ref,
          q20_ref, bq20_ref, q21_ref, bq21_ref,
          k20_ref, bk20_ref, k21_ref, bk21_ref,
          v20_ref, bv20_ref, v21_ref, bv21_ref,
          oW_ref, ob_ref, eyeB_ref, out_ref):
    for g in range(GPB):
        feats = feats_ref[g]
        h = jnp.maximum(feats @ W1_ref[:] + b1_ref[:], 0.0)
        h = jnp.maximum(h @ W2_ref[:] + b2_ref[:], 0.0)

        pos = pos_ref[g]
        posT = posT_ref[g]
        dx = pos[:, 0:1] - posT[0:1, :]
        dy = pos[:, 1:2] - posT[1:2, :]
        d2 = dx * dx + dy * dy
        t = jnp.maximum(d2 + eyeB_ref[:], 0.0)
        mask_add = (t * -1e38).astype(jnp.bfloat16)

        h = jnp.maximum(
            _attn(h.astype(jnp.bfloat16), mask_add,
                  (q10_ref, q11_ref), (bq10_ref, bq11_ref),
                  (k10_ref, k11_ref), (bk10_ref, bk11_ref),
                  (v10_ref, v11_ref), (bv10_ref, bv11_ref)), 0.0)
        ohs = _attn(h.astype(jnp.bfloat16), mask_add,
                    (q20_ref, q21_ref), (bq20_ref, bq21_ref),
                    (k20_ref, k21_ref), (bk20_ref, bk21_ref),
                    (v20_ref, v21_ref), (bv20_ref, bv21_ref),
                    normalize=False)
        embs = []
        for oh in ohs:
            gh = jax.lax.dot_general(
                onehot_ref[g], jnp.maximum(oh, 0.0), (((1,), (0,)), ((), ())),
                preferred_element_type=jnp.float32,
            )
            embs.append(gh[:, :HIDDEN] / (gh[:, HIDDEN:HE] + 1e-16))
        emb = jnp.concatenate(embs, axis=1)
        out_ref[g] = emb @ oW_ref[:] + ob_ref[:]


def _split_heads(Wq, bq, Wk, bk, Wv, bv):
    scale = 1.0 / math.sqrt(HIDDEN)
    out = []
    for h in range(HEADS):
        sl = slice(h * HIDDEN, (h + 1) * HIDDEN)
        wqh = (Wq[:, sl] * scale).astype(jnp.bfloat16)
        bqh = (bq[sl] * scale)[None, :]
        wkh = Wk[:, sl].astype(jnp.bfloat16)
        bkh = bk[sl][None, :]
        wvh = jnp.concatenate(
            [Wv[:, sl], jnp.zeros((Wv.shape[0], 1), jnp.float32)], axis=1
        ).astype(jnp.bfloat16)
        bvh = jnp.concatenate([bv[sl], jnp.ones((1,), jnp.float32)])[None, :]
        out.extend([wqh, bqh, wkh, bkh, wvh, bvh])
    return [out[i] for i in (0, 1, 6, 7, 2, 3, 8, 9, 4, 5, 10, 11)]


@jax.jit
def kernel(obs, enc_W1, enc_b1, enc_W2, enc_b2,
           c1_Wq, c1_bq, c1_Wk, c1_bk, c1_Wv, c1_bv,
           c2_Wq, c2_bq, c2_Wk, c2_bk, c2_Wv, c2_bv,
           out_W, out_b):
    node = obs[:, :N * (2 + INPUT_DIM)].reshape(BS, N, 2 + INPUT_DIM)
    pos = node[:, :, :2]
    feats = node[:, :, 2:]
    pos_p = jnp.pad(pos, ((0, 0), (0, NP - N), (0, 0)), constant_values=1e6)
    feats_p = jnp.pad(feats, ((0, 0), (0, NP - N), (0, 2)))
    posT_p = jnp.swapaxes(pos_p, 1, 2)

    agent = jnp.clip(obs[:, -1], 0, N - 1).astype(jnp.int32)
    onehot = jax.nn.one_hot(agent, NP, dtype=jnp.float32)[:, None, :]

    W1 = jnp.pad(enc_W1, ((0, 2), (0, 0)))
    b1 = enc_b1[None, :]
    b2 = enc_b2[None, :]
    ob = out_b[None, :]
    l1 = _split_heads(c1_Wq, c1_bq, c1_Wk, c1_bk, c1_Wv, c1_bv)
    l2 = _split_heads(c2_Wq, c2_bq, c2_Wk, c2_bk, c2_Wv, c2_bv)

    def fixed(a):
        nd = a.ndim
        return pl.BlockSpec(a.shape, lambda b: (0,) * nd)

    r2 = jnp.float32(RADIUS * RADIUS)
    eyeB = jnp.full((NP, NP), -r2, jnp.float32) + jnp.eye(NP, dtype=jnp.float32) * (1.0 + r2)
    weights = [W1, b1, enc_W2, b2] + l1 + l2 + [out_W, ob, eyeB]
    in_specs = [
        pl.BlockSpec((GPB, NP, 8), lambda b: (b, 0, 0)),
        pl.BlockSpec((GPB, NP, 2), lambda b: (b, 0, 0)),
        pl.BlockSpec((GPB, 2, NP), lambda b: (b, 0, 0)),
        pl.BlockSpec((GPB, 1, NP), lambda b: (b, 0, 0)),
    ] + [fixed(w) for w in weights]

    out = pl.pallas_call(
        _body,
        grid=(BS // GPB,),
        in_specs=in_specs,
        out_specs=pl.BlockSpec((GPB, 1, OUT_DIM), lambda b: (b, 0, 0)),
        out_shape=jax.ShapeDtypeStruct((BS, 1, OUT_DIM), jnp.float32),
    )(feats_p, pos_p, posT_p, onehot, *weights)
    return out[:, 0, :]
